# Initial kernel scaffold; baseline (speedup 1.0000x reference)
import jax
import jax.numpy as jnp
from jax.experimental import pallas as pl


def kernel(x, rel_feat, batch_heads, batch_rels, batch_tails, batch_ids, weights):
    raise NotImplementedError("write your pallas kernel here")


# speedup vs baseline: 2.5986x; 2.5986x over previous
import functools

import jax
import jax.numpy as jnp
from jax import lax
from jax.experimental import pallas as pl
from jax.experimental.pallas import tpu as pltpu
from jax.experimental.pallas import tpu_sc as plsc

N_ENT = 10000
NUM_REL = 200
BATCH = 10
N_FACT = 320000
D = 128

NC, NS, L = 2, 16, 16
NW = NC * NS
CK = 128
CHUNKS_PER_TILE = 80
NPAD = NW * CHUNKS_PER_TILE * CK
NROW = N_ENT + BATCH * NUM_REL
STRIPE = 752


def _sc_gnn(x, rel_feat, heads, rels, ids, tails, w):
    mesh = plsc.VectorSubcoreMesh(core_axis_name="c", subcore_axis_name="s")

    @functools.partial(
        pl.kernel,
        out_type=jax.ShapeDtypeStruct((NC, NROW, D), jnp.float32),
        mesh=mesh,
        scratch_types=[
            pltpu.VMEM_SHARED((NROW, D), jnp.float32),
            pltpu.VMEM((CK, D), jnp.float32),
            pltpu.VMEM((CK, D), jnp.float32),
            pltpu.VMEM((CK,), jnp.int32),
            pltpu.VMEM((CK,), jnp.int32),
            pltpu.VMEM((CK,), jnp.int32),
            pltpu.VMEM((CK,), jnp.int32),
            pltpu.VMEM((CK,), jnp.int32),
            pltpu.VMEM((CK,), jnp.int32),
            pltpu.VMEM((CK + L,), jnp.float32),
            pltpu.SemaphoreType.DMA,
        ],
    )
    def body(x_h, rf_h, hd_h, rl_h, id_h, tl_h, w_h, out_h,
             acc, xrows, rrows, hv, rv, iv, tv, riv, rsv, w2v, sem):
        cid = lax.axis_index("c")
        sid = lax.axis_index("s")
        wid = cid * NS + sid

        zvec = jnp.zeros((L,), jnp.float32)

        def zrow(r, carry):
            for j in range(D // L):
                xrows[r, pl.ds(L * j, L)] = zvec
            return carry

        lax.fori_loop(0, CK, zrow, 0)
        sbase = jnp.minimum(sid * STRIPE, NROW - STRIPE)
        for kk in range(STRIPE // CK):
            pltpu.sync_copy(xrows, acc.at[pl.ds(sbase + kk * CK, CK)])
        rem = STRIPE % CK
        if rem:
            pltpu.sync_copy(
                xrows.at[pl.ds(0, rem)],
                acc.at[pl.ds(sbase + (STRIPE // CK) * CK, rem)],
            )
        plsc.subcore_barrier()

        def chunk(k, carry):
            base = (wid * CHUNKS_PER_TILE + k) * CK
            pltpu.sync_copy(hd_h.at[pl.ds(base, CK)], hv)
            pltpu.sync_copy(rl_h.at[pl.ds(base, CK)], rv)
            pltpu.sync_copy(id_h.at[pl.ds(base, CK)], iv)
            pltpu.sync_copy(tl_h.at[pl.ds(base, CK)], tv)
            pltpu.sync_copy(w_h.at[pl.ds(base, CK)], w2v.at[pl.ds(0, CK)])
            for j in range(CK // L):
                sl = pl.ds(L * j, L)
                r16 = rv[sl] + iv[sl] * NUM_REL
                riv[sl] = r16
                rsv[sl] = r16 + N_ENT
                w16 = w2v[sl]
                w2v[sl] = w16 * w16
            pltpu.async_copy(x_h.at[hv], xrows, sem).wait()
            pltpu.async_copy(rf_h.at[riv], rrows, sem).wait()

            def fact(f, c2):
                s = w2v[pl.ds(f, L)][0]
                for j in range(D // L):
                    sl = pl.ds(L * j, L)
                    xrows[f, sl] = (xrows[f, sl] + rrows[f, sl]) * s
                return c2

            lax.fori_loop(0, CK, fact, 0)
            pltpu.sync_copy(xrows, acc.at[tv], add=True)
            pltpu.sync_copy(xrows, acc.at[rsv], add=True)
            return carry

        lax.fori_loop(0, CHUNKS_PER_TILE, chunk, 0)
        plsc.subcore_barrier()
        pltpu.sync_copy(
            acc.at[pl.ds(sbase, STRIPE)],
            out_h.at[cid, pl.ds(sbase, STRIPE)],
        )

    return body(x, rel_feat, heads, rels, ids, tails, w)


def _tc_reduce(parts):
    BR = 1000

    def red(p_ref, o_ref):
        o_ref[...] = p_ref[0] + p_ref[1]

    return pl.pallas_call(
        red,
        grid=(NROW // BR,),
        in_specs=[pl.BlockSpec((NC, BR, D), lambda i: (0, i, 0))],
        out_specs=pl.BlockSpec((BR, D), lambda i: (i, 0)),
        out_shape=jax.ShapeDtypeStruct((NROW, D), jnp.float32),
    )(parts)


def kernel(x, rel_feat, batch_heads, batch_rels, batch_tails, batch_ids, weights):
    pad = NPAD - N_FACT
    zi = jnp.zeros((pad,), jnp.int32)
    heads = jnp.concatenate([batch_heads, zi])
    rels = jnp.concatenate([batch_rels, zi])
    ids = jnp.concatenate([batch_ids, zi])
    tails = jnp.concatenate([batch_tails, zi])
    w = jnp.concatenate([weights, jnp.zeros((pad,), jnp.float32)])
    parts = _sc_gnn(x, rel_feat, heads, rels, ids, tails, w)
    summed = _tc_reduce(parts)
    return summed[:N_ENT], summed[N_ENT:]


# speedup vs baseline: 4.5892x; 1.7660x over previous
import functools

import jax
import jax.numpy as jnp
from jax import lax
from jax.experimental import pallas as pl
from jax.experimental.pallas import tpu as pltpu
from jax.experimental.pallas import tpu_sc as plsc

N_ENT = 10000
NUM_REL = 200
BATCH = 10
N_FACT = 320000
D = 128

NC, NS, L = 2, 16, 16
NW = NC * NS
CK = 48
NCH = 210
NPAD = NW * NCH * CK
NROW = N_ENT + BATCH * NUM_REL
STRIPE = 752


def _sc_gnn(x, rel_feat, heads, rels, ids, tails, w):
    mesh = plsc.VectorSubcoreMesh(core_axis_name="c", subcore_axis_name="s")

    def buffer_set():
        return [
            pltpu.VMEM((CK, D), jnp.float32),
            pltpu.VMEM((CK, D), jnp.float32),
            pltpu.VMEM((CK,), jnp.int32),
            pltpu.VMEM((CK,), jnp.int32),
            pltpu.VMEM((CK,), jnp.int32),
            pltpu.VMEM((CK,), jnp.int32),
            pltpu.VMEM((CK,), jnp.int32),
            pltpu.VMEM((CK,), jnp.int32),
            pltpu.VMEM((CK + L,), jnp.float32),
            pltpu.SemaphoreType.DMA,
            pltpu.SemaphoreType.DMA,
        ]

    @functools.partial(
        pl.kernel,
        out_type=jax.ShapeDtypeStruct((NC, NROW, D), jnp.float32),
        mesh=mesh,
        scratch_types=[pltpu.VMEM_SHARED((NROW, D), jnp.float32)]
        + buffer_set() + buffer_set(),
    )
    def body(x_h, rf_h, hd_h, rl_h, id_h, tl_h, w_h, out_h, acc, *bufs):
        sets = (bufs[:11], bufs[11:])
        cid = lax.axis_index("c")
        sid = lax.axis_index("s")
        wid = cid * NS + sid

        def prep(c, bset):
            _, _, hv, rv, iv, tv, riv, rsv, w2v, sem_i, _ = bset
            base = (wid * NCH + c) * CK
            cps = [
                pltpu.async_copy(hd_h.at[pl.ds(base, CK)], hv, sem_i),
                pltpu.async_copy(rl_h.at[pl.ds(base, CK)], rv, sem_i),
                pltpu.async_copy(id_h.at[pl.ds(base, CK)], iv, sem_i),
                pltpu.async_copy(tl_h.at[pl.ds(base, CK)], tv, sem_i),
                pltpu.async_copy(w_h.at[pl.ds(base, CK)], w2v.at[pl.ds(0, CK)], sem_i),
            ]
            for cp in cps:
                cp.wait()
            for j in range(CK // L):
                sl = pl.ds(L * j, L)
                r16 = rv[sl] + iv[sl] * NUM_REL
                riv[sl] = r16
                rsv[sl] = r16 + N_ENT
                w16 = w2v[sl]
                w2v[sl] = w16 * w16

        def gstart(bset):
            xrows, rrows, hv, _, _, _, riv = bset[0], bset[1], bset[2], bset[3], bset[4], bset[5], bset[6]
            sem_g = bset[10]
            pltpu.async_copy(x_h.at[hv], xrows, sem_g)
            pltpu.async_copy(rf_h.at[riv], rrows, sem_g)

        def gwait(bset):
            xrows, rrows, hv, riv = bset[0], bset[1], bset[2], bset[6]
            sem_g = bset[10]
            pltpu.make_async_copy(x_h.at[hv], xrows, sem_g).wait()
            pltpu.make_async_copy(rf_h.at[riv], rrows, sem_g).wait()

        def compute(bset):
            xrows, rrows, w2v = bset[0], bset[1], bset[8]

            def fact(f, c2):
                s = w2v[pl.ds(f, L)][0]
                for j in range(D // L):
                    sl = pl.ds(L * j, L)
                    xrows[f, sl] = (xrows[f, sl] + rrows[f, sl]) * s
                return c2

            lax.fori_loop(0, CK, fact, 0, unroll=2)

        def scatter(bset):
            xrows, tv, rsv = bset[0], bset[5], bset[7]
            pltpu.sync_copy(xrows, acc.at[tv], add=True)
            pltpu.sync_copy(xrows, acc.at[rsv], add=True)

        xrows0 = sets[0][0]
        zvec = jnp.zeros((L,), jnp.float32)

        def zrow(r, carry):
            for j in range(D // L):
                xrows0[r, pl.ds(L * j, L)] = zvec
            return carry

        lax.fori_loop(0, CK, zrow, 0)
        sbase = jnp.minimum(sid * STRIPE, NROW - STRIPE)
        for kk in range(STRIPE // CK):
            pltpu.sync_copy(xrows0, acc.at[pl.ds(sbase + kk * CK, CK)])
        rem = STRIPE % CK
        if rem:
            pltpu.sync_copy(
                xrows0.at[pl.ds(0, rem)],
                acc.at[pl.ds(sbase + (STRIPE // CK) * CK, rem)],
            )
        plsc.subcore_barrier()

        prep(0, sets[0])
        gstart(sets[0])
        prep(1, sets[1])
        gstart(sets[1])

        def pair(p, carry):
            c0 = 2 * p
            for half, bset in enumerate(sets):
                c = c0 + half
                gwait(bset)
                compute(bset)
                scatter(bset)
                cpre = jnp.minimum(c + 2, NCH - 2 + half)
                prep(cpre, bset)
                gstart(bset)
            return carry

        lax.fori_loop(0, NCH // 2, pair, 0)
        gwait(sets[0])
        gwait(sets[1])

        plsc.subcore_barrier()
        pltpu.sync_copy(
            acc.at[pl.ds(sbase, STRIPE)],
            out_h.at[cid, pl.ds(sbase, STRIPE)],
        )

    return body(x, rel_feat, heads, rels, ids, tails, w)


def _tc_reduce(parts):
    BR = 1000

    def red(p_ref, o_ref):
        o_ref[...] = p_ref[0] + p_ref[1]

    return pl.pallas_call(
        red,
        grid=(NROW // BR,),
        in_specs=[pl.BlockSpec((NC, BR, D), lambda i: (0, i, 0))],
        out_specs=pl.BlockSpec((BR, D), lambda i: (i, 0)),
        out_shape=jax.ShapeDtypeStruct((NROW, D), jnp.float32),
    )(parts)


def kernel(x, rel_feat, batch_heads, batch_rels, batch_tails, batch_ids, weights):
    pad = NPAD - N_FACT
    zi = jnp.zeros((pad,), jnp.int32)
    heads = jnp.concatenate([batch_heads, zi])
    rels = jnp.concatenate([batch_rels, zi])
    ids = jnp.concatenate([batch_ids, zi])
    tails = jnp.concatenate([batch_tails, zi])
    w = jnp.concatenate([weights, jnp.zeros((pad,), jnp.float32)])
    parts = _sc_gnn(x, rel_feat, heads, rels, ids, tails, w)
    summed = _tc_reduce(parts)
    return summed[:N_ENT], summed[N_ENT:]


# speedup vs baseline: 5.1737x; 1.1274x over previous
import functools

import jax
import jax.numpy as jnp
from jax import lax
from jax.experimental import pallas as pl
from jax.experimental.pallas import tpu as pltpu
from jax.experimental.pallas import tpu_sc as plsc

N_ENT = 10000
NUM_REL = 200
BATCH = 10
N_FACT = 320000
D = 128

NC, NS, L = 2, 16, 16
NW = NC * NS
CK = 48
NCH = 210
NPAD = NW * NCH * CK
NROW = N_ENT + BATCH * NUM_REL
STRIPE = 752


def _sc_gnn(x, rel_feat, heads, rels, ids, tails, w):
    mesh = plsc.VectorSubcoreMesh(core_axis_name="c", subcore_axis_name="s")

    def buffer_set():
        return [
            pltpu.VMEM((CK, D), jnp.float32),
            pltpu.VMEM((CK, D), jnp.float32),
            pltpu.VMEM((CK,), jnp.int32),
            pltpu.VMEM((CK,), jnp.int32),
            pltpu.VMEM((CK,), jnp.int32),
            pltpu.VMEM((CK,), jnp.int32),
            pltpu.VMEM((CK,), jnp.float32),
            pltpu.VMEM((CK,), jnp.int32),
            pltpu.VMEM((CK,), jnp.int32),
            pltpu.VMEM((CK,), jnp.int32),
            pltpu.VMEM((CK + L,), jnp.float32),
            pltpu.SemaphoreType.DMA,
            pltpu.SemaphoreType.DMA,
        ]

    @functools.partial(
        pl.kernel,
        out_type=jax.ShapeDtypeStruct((NC, NROW, D), jnp.float32),
        mesh=mesh,
        scratch_types=[pltpu.VMEM_SHARED((NROW, D), jnp.float32)]
        + buffer_set() + buffer_set(),
    )
    def body(x_h, rf_h, hd_h, rl_h, id_h, tl_h, w_h, out_h, acc, *bufs):
        sets = (bufs[:13], bufs[13:])
        cid = lax.axis_index("c")
        sid = lax.axis_index("s")
        wid = cid * NS + sid

        def idx_copies(c, bset):
            hv, rv, iv, tl, wraw, sem_i = bset[2], bset[3], bset[4], bset[5], bset[6], bset[11]
            base = (wid * NCH + c) * CK
            return [
                pltpu.async_copy(hd_h.at[pl.ds(base, CK)], hv, sem_i),
                pltpu.async_copy(rl_h.at[pl.ds(base, CK)], rv, sem_i),
                pltpu.async_copy(id_h.at[pl.ds(base, CK)], iv, sem_i),
                pltpu.async_copy(tl_h.at[pl.ds(base, CK)], tl, sem_i),
                pltpu.async_copy(w_h.at[pl.ds(base, CK)], wraw, sem_i),
            ]

        def idx_issue(c, bset):
            idx_copies(c, bset)

        def idx_wait(c, bset):
            hv, rv, iv, tl, wraw, sem_i = bset[2], bset[3], bset[4], bset[5], bset[6], bset[11]
            base = (wid * NCH + c) * CK
            pltpu.make_async_copy(hd_h.at[pl.ds(base, CK)], hv, sem_i).wait()
            pltpu.make_async_copy(rl_h.at[pl.ds(base, CK)], rv, sem_i).wait()
            pltpu.make_async_copy(id_h.at[pl.ds(base, CK)], iv, sem_i).wait()
            pltpu.make_async_copy(tl_h.at[pl.ds(base, CK)], tl, sem_i).wait()
            pltpu.make_async_copy(w_h.at[pl.ds(base, CK)], wraw, sem_i).wait()

        def vec_prep(bset):
            rv, iv, tl, wraw, riv, rsv, tv, w2v = (
                bset[3], bset[4], bset[5], bset[6], bset[7], bset[8], bset[9], bset[10])
            for j in range(CK // L):
                sl = pl.ds(L * j, L)
                r16 = rv[sl] + iv[sl] * NUM_REL
                riv[sl] = r16
                rsv[sl] = r16 + N_ENT
                tv[sl] = tl[sl]
                w16 = wraw[sl]
                w2v[sl] = w16 * w16

        def gstart(bset):
            xrows, rrows, hv, riv, sem_g = bset[0], bset[1], bset[2], bset[7], bset[12]
            pltpu.async_copy(x_h.at[hv], xrows, sem_g)
            pltpu.async_copy(rf_h.at[riv], rrows, sem_g)

        def gwait(bset):
            xrows, rrows, hv, riv, sem_g = bset[0], bset[1], bset[2], bset[7], bset[12]
            pltpu.make_async_copy(x_h.at[hv], xrows, sem_g).wait()
            pltpu.make_async_copy(rf_h.at[riv], rrows, sem_g).wait()

        def compute(bset):
            xrows, rrows, w2v = bset[0], bset[1], bset[10]

            def fact(f, c2):
                s = w2v[pl.ds(f, L)][0]
                for j in range(D // L):
                    sl = pl.ds(L * j, L)
                    xrows[f, sl] = (xrows[f, sl] + rrows[f, sl]) * s
                return c2

            lax.fori_loop(0, CK, fact, 0, unroll=4)

        def scatter(bset):
            xrows, rsv, tv = bset[0], bset[8], bset[9]
            pltpu.sync_copy(xrows, acc.at[tv], add=True)
            pltpu.sync_copy(xrows, acc.at[rsv], add=True)

        xrows0 = sets[0][0]
        zvec = jnp.zeros((L,), jnp.float32)

        def zrow(r, carry):
            for j in range(D // L):
                xrows0[r, pl.ds(L * j, L)] = zvec
            return carry

        lax.fori_loop(0, CK, zrow, 0)
        sbase = jnp.minimum(sid * STRIPE, NROW - STRIPE)
        for kk in range(STRIPE // CK):
            pltpu.sync_copy(xrows0, acc.at[pl.ds(sbase + kk * CK, CK)])
        rem = STRIPE % CK
        if rem:
            pltpu.sync_copy(
                xrows0.at[pl.ds(0, rem)],
                acc.at[pl.ds(sbase + (STRIPE // CK) * CK, rem)],
            )
        plsc.subcore_barrier()

        for half, bset in enumerate(sets):
            idx_issue(half, bset)
            idx_wait(half, bset)
            vec_prep(bset)
            gstart(bset)

        def pair(p, carry):
            c0 = 2 * p
            for half, bset in enumerate(sets):
                c = c0 + half
                cpre = jnp.minimum(c + 2, NCH - 2 + half)
                gwait(bset)
                idx_issue(cpre, bset)
                compute(bset)
                scatter(bset)
                idx_wait(cpre, bset)
                vec_prep(bset)
                gstart(bset)
            return carry

        lax.fori_loop(0, NCH // 2, pair, 0)
        gwait(sets[0])
        gwait(sets[1])

        plsc.subcore_barrier()
        pltpu.sync_copy(
            acc.at[pl.ds(sbase, STRIPE)],
            out_h.at[cid, pl.ds(sbase, STRIPE)],
        )

    return body(x, rel_feat, heads, rels, ids, tails, w)


def _tc_reduce(parts):
    BR = 1000

    def red(p_ref, o_ref):
        o_ref[...] = p_ref[0] + p_ref[1]

    return pl.pallas_call(
        red,
        grid=(NROW // BR,),
        in_specs=[pl.BlockSpec((NC, BR, D), lambda i: (0, i, 0))],
        out_specs=pl.BlockSpec((BR, D), lambda i: (i, 0)),
        out_shape=jax.ShapeDtypeStruct((NROW, D), jnp.float32),
    )(parts)


def kernel(x, rel_feat, batch_heads, batch_rels, batch_tails, batch_ids, weights):
    pad = NPAD - N_FACT
    zi = jnp.zeros((pad,), jnp.int32)
    heads = jnp.concatenate([batch_heads, zi])
    rels = jnp.concatenate([batch_rels, zi])
    ids = jnp.concatenate([batch_ids, zi])
    tails = jnp.concatenate([batch_tails, zi])
    w = jnp.concatenate([weights, jnp.zeros((pad,), jnp.float32)])
    parts = _sc_gnn(x, rel_feat, heads, rels, ids, tails, w)
    summed = _tc_reduce(parts)
    return summed[:N_ENT], summed[N_ENT:]
